# baseline (speedup 1.0000x reference)
import jax, jax.numpy as jnp
import numpy as np
from jax.experimental import pallas as pl

DIM = [64, 128, 256]
LIST_K = [10, 6, 3, 2]
RATIO = 1.0 / 6.0
K0 = 10


def pairwise_topk(query, base, k, chunk=2000):
    Q, D = query.shape
    pad = (-Q) % chunk
    qp = jnp.pad(query, ((0, pad), (0, 0)))
    qp = qp.reshape(-1, chunk, D)
    def f(qc):
        d = jnp.sum((qc[:, None, :] - base[None, :, :]) ** 2, -1)
        return jax.lax.top_k(-d, k)[1]
    idx = jax.lax.map(f, qp).reshape(-1, k)
    return idx[:Q]


def knn_graph(pos, k):
    idx = pairwise_topk(pos, pos, k + 1)[:, 1:]
    n = pos.shape[0]
    return idx.reshape(-1), jnp.repeat(jnp.arange(n), k)


def fps(pos, ratio):
    n = pos.shape[0]
    m = max(1, int(n * ratio))
    d0 = jnp.sum((pos - pos[0]) ** 2, -1)
    sel0 = jnp.zeros((m,), jnp.int32)
    def body(i, c):
        sel, dist = c
        idx = jnp.argmax(dist).astype(jnp.int32)
        sel = sel.at[i].set(idx)
        nd = jnp.sum((pos - pos[idx]) ** 2, -1)
        return sel, jnp.minimum(dist, nd)
    sel, _ = jax.lax.fori_loop(1, m, body, (sel0, d0))
    return sel


def build_structs(pos):
    src0, dst0 = knn_graph(pos, K0)
    levels = [{'pos': pos, 'src': src0, 'dst': dst0, 'n': pos.shape[0]}]
    tds = []
    cur = pos
    for i in range(2):
        idc = fps(cur, RATIO)
        m = idc.shape[0]
        sub = cur[idc]
        nbr = pairwise_topk(sub, cur, LIST_K[i])
        es, ed = knn_graph(sub, LIST_K[i + 1])
        tds.append({'nbr': nbr.reshape(-1), 'seg': jnp.repeat(jnp.arange(m), LIST_K[i]), 'm': m})
        levels.append({'pos': sub, 'src': es, 'dst': ed, 'n': m})
        cur = sub
    ups = []
    for i in range(2):
        idx = pairwise_topk(levels[i]['pos'], levels[i + 1]['pos'], 2)
        d = jnp.sum((levels[i]['pos'][:, None, :] - levels[i + 1]['pos'][idx]) ** 2, -1)
        w = 1.0 / (d + 1e-8)
        w = w / jnp.sum(w, -1, keepdims=True)
        ups.append({'idx': idx, 'w': w})
    return levels, tds, ups


def layer_norm(x):
    m = x.mean(-1, keepdims=True)
    v = ((x - m) ** 2).mean(-1, keepdims=True)
    return (x - m) / jnp.sqrt(v + 1e-5)


def mlp_gn(x, p):
    h = x @ p['w'] + p['b']
    mu = h.mean(0, keepdims=True)
    var = ((h - mu) ** 2).mean(0, keepdims=True)
    return jax.nn.relu((h - mu) / jnp.sqrt(var + 1e-5))


def genconv(p, x, src, dst, n):
    m = jax.nn.relu(x[src]) + 1e-7
    mt = m * p['t']
    mx = jax.ops.segment_max(mt, dst, num_segments=n)
    mx = jnp.where(jnp.isfinite(mx), mx, 0.0)
    e = jnp.exp(mt - mx[dst])
    s = jax.ops.segment_sum(e, dst, num_segments=n)
    alpha = e / (s[dst] + 1e-16)
    aggr = jax.ops.segment_sum(alpha * m, dst, num_segments=n)
    h = x + aggr
    h = jax.nn.relu(layer_norm(h @ p['w1'] + p['b1']))
    return h @ p['w2'] + p['b2']


def deepgcn(p, x, src, dst, n):
    h = jax.nn.relu(layer_norm(x))
    return x + genconv(p, h, src, dst, n)


def mha(x, p, nh):
    B, S, D = x.shape
    dh = D // nh
    def sp(y):
        return y.reshape(B, S, nh, dh).transpose(0, 2, 1, 3)
    q = sp(x @ p['wq']['w'] + p['wq']['b'])
    k = sp(x @ p['wk']['w'] + p['wk']['b'])
    v = sp(x @ p['wv']['w'] + p['wv']['b'])
    a = jax.nn.softmax(q @ k.transpose(0, 1, 3, 2) / np.sqrt(dh), -1)
    o = (a @ v).transpose(0, 2, 1, 3).reshape(B, S, D)
    return o @ p['wo']['w'] + p['wo']['b']


def tlayer(x, p, nh=2):
    x = layer_norm(x + mha(x, p, nh))
    f = jax.nn.relu(x @ p['f1']['w'] + p['f1']['b']) @ p['f2']['w'] + p['f2']['b']
    return layer_norm(x + f)


def spenc_feats(pos):
    scales = jnp.geomspace(1e-6, 8.0, 4).astype(jnp.float32)
    f = pos[:, None, :] / scales[None, :, None]
    return jnp.concatenate([jnp.sin(f), jnp.cos(f)], -1).reshape(pos.shape[0], -1)


def forward_core(x, P, levels, tds, ups):
    pos = levels[0]['pos']
    emb = jax.nn.relu(spenc_feats(pos) @ P['spenc']['w'] + P['spenc']['b'])
    emb = jnp.tanh(emb @ P['dec1']['w'] + P['dec1']['b'])
    emb = emb @ P['dec2']['w'] + P['dec2']['b']
    x = jnp.concatenate([x, emb], 1)
    x = mlp_gn(x, P['mlp_in'])
    l0 = levels[0]
    x = genconv(P['input_gc'][0], x, l0['src'], l0['dst'], l0['n'])
    x = deepgcn(P['input_gc'][1], x, l0['src'], l0['dst'], l0['n'])
    outs = [x]
    for i in range(2):
        h = mlp_gn(x, P['td_mlp'][i])
        x = jax.ops.segment_max(h[tds[i]['nbr']], tds[i]['seg'], num_segments=tds[i]['m'])
        l = levels[i + 1]
        x = jax.nn.relu(deepgcn(P['down_gc'][i], x, l['src'], l['dst'], l['n']))
        x = jax.nn.relu(deepgcn(P['down_gc'][i], x, l['src'], l['dst'], l['n']))
        outs.append(x)
    xb = jax.nn.relu(x[None] @ P['mlp_summit']['w'] + P['mlp_summit']['b'])
    for tp in P['summit_tf']:
        xb = tlayer(xb, tp)
    x = xb[0]
    for i in range(2):
        lvl = 1 - i
        up = ups[lvl]
        xs = mlp_gn(x, P['tu_mlp_sub'][lvl])
        interp = jnp.sum(xs[up['idx']] * up['w'][..., None], 1)
        x = mlp_gn(outs[lvl], P['tu_mlp'][lvl]) + interp
        l = levels[lvl]
        x = jax.nn.relu(deepgcn(P['up_gc'][lvl], x, l['src'], l['dst'], l['n']))
        x = jax.nn.relu(deepgcn(P['up_gc'][lvl], x, l['src'], l['dst'], l['n']))
    for gp in P['out_gc']:
        x = deepgcn(gp, x, l0['src'], l0['dst'], l0['n'])
    return x @ P['mlp_out']['w'] + P['mlp_out']['b']


def _identity_body(x_ref, o_ref):
    o_ref[...] = x_ref[...]


def _pallas_identity(x):
    n, d = x.shape
    npad = -(-n // 8) * 8
    xp = jnp.pad(x, ((0, npad - n), (0, 0)))
    out = pl.pallas_call(
        _identity_body,
        out_shape=jax.ShapeDtypeStruct((npad, d), jnp.float32),
    )(xp)
    return out[:n]


def kernel(x, pos, batch, params):
    levels, tds, ups = build_structs(pos)
    out = forward_core(x, params, levels, tds, ups)
    return _pallas_identity(out)


# speedup vs baseline: 1.3512x; 1.3512x over previous
import functools

import jax
import jax.numpy as jnp
import numpy as np
from jax.experimental import pallas as pl
from jax.experimental.pallas import tpu as pltpu

_DIM = [64, 128, 256]
_LIST_K = [10, 6, 3, 2]
_RATIO = 1.0 / 6.0
_K0 = 10


def _fps_body(px_ref, py_ref, pz_ref, o_ref, dist_ref, *, n, m, rows):
    iota = (jax.lax.broadcasted_iota(jnp.int32, (rows, 128), 0) * 128
            + jax.lax.broadcasted_iota(jnp.int32, (rows, 128), 1))
    valid = iota < n
    big = jnp.int32(2 ** 30)

    def point_at(fidx):
        mask = (iota == fidx)
        qx = jnp.sum(jnp.where(mask, px_ref[...], 0.0))
        qy = jnp.sum(jnp.where(mask, py_ref[...], 0.0))
        qz = jnp.sum(jnp.where(mask, pz_ref[...], 0.0))
        return qx, qy, qz

    def dist_to(qx, qy, qz):
        dx = px_ref[...] - qx
        dy = py_ref[...] - qy
        dz = pz_ref[...] - qz
        return dx * dx + dy * dy + dz * dz

    qx, qy, qz = point_at(jnp.int32(0))
    d0 = dist_to(qx, qy, qz)
    dist_ref[...] = jnp.where(valid, d0, -1.0)
    o_ref[0, 0] = jnp.int32(0)

    def body(i, _):
        dist = dist_ref[...]
        mval = jnp.max(dist)
        fidx = jnp.min(jnp.where(dist == mval, iota, big))
        o_ref[0, i] = fidx
        qx, qy, qz = point_at(fidx)
        nd = dist_to(qx, qy, qz)
        dist_ref[...] = jnp.minimum(dist, nd)
        return 0

    jax.lax.fori_loop(1, m, body, 0)


def _fps_pallas(pos, ratio):
    n = pos.shape[0]
    m = max(1, int(n * ratio))
    rows = -(-n // 128)
    npad = rows * 128
    posp = jnp.pad(pos, ((0, npad - n), (0, 0)))
    px = posp[:, 0].reshape(rows, 128)
    py = posp[:, 1].reshape(rows, 128)
    pz = posp[:, 2].reshape(rows, 128)
    sel = pl.pallas_call(
        functools.partial(_fps_body, n=n, m=m, rows=rows),
        in_specs=[pl.BlockSpec((rows, 128), lambda: (0, 0))] * 3,
        out_specs=pl.BlockSpec(memory_space=pltpu.SMEM),
        out_shape=jax.ShapeDtypeStruct((1, m), jnp.int32),
        scratch_shapes=[pltpu.VMEM((rows, 128), jnp.float32)],
    )(px, py, pz)
    return sel[0]


def _pairwise_topk(query, base, k, chunk=2000):
    q, d = query.shape
    pad = (-q) % chunk
    qp = jnp.pad(query, ((0, pad), (0, 0)))
    qp = qp.reshape(-1, chunk, d)

    def f(qc):
        dist = jnp.sum((qc[:, None, :] - base[None, :, :]) ** 2, -1)
        return jax.lax.top_k(-dist, k)[1]

    idx = jax.lax.map(f, qp).reshape(-1, k)
    return idx[:q]


def _knn_graph(pos, k):
    idx = _pairwise_topk(pos, pos, k + 1)[:, 1:]
    n = pos.shape[0]
    return idx.reshape(-1), jnp.repeat(jnp.arange(n), k)


def _build_structs(pos):
    src0, dst0 = _knn_graph(pos, _K0)
    levels = [{'pos': pos, 'src': src0, 'dst': dst0, 'n': pos.shape[0]}]
    tds = []
    cur = pos
    for i in range(2):
        idc = _fps_pallas(cur, _RATIO)
        m = idc.shape[0]
        sub = cur[idc]
        nbr = _pairwise_topk(sub, cur, _LIST_K[i])
        es, ed = _knn_graph(sub, _LIST_K[i + 1])
        tds.append({'nbr': nbr.reshape(-1),
                    'seg': jnp.repeat(jnp.arange(m), _LIST_K[i]), 'm': m})
        levels.append({'pos': sub, 'src': es, 'dst': ed, 'n': m})
        cur = sub
    ups = []
    for i in range(2):
        idx = _pairwise_topk(levels[i]['pos'], levels[i + 1]['pos'], 2)
        d = jnp.sum((levels[i]['pos'][:, None, :]
                     - levels[i + 1]['pos'][idx]) ** 2, -1)
        w = 1.0 / (d + 1e-8)
        w = w / jnp.sum(w, -1, keepdims=True)
        ups.append({'idx': idx, 'w': w})
    return levels, tds, ups


def _layer_norm(x):
    m = x.mean(-1, keepdims=True)
    v = ((x - m) ** 2).mean(-1, keepdims=True)
    return (x - m) / jnp.sqrt(v + 1e-5)


def _mlp_gn(x, p):
    h = x @ p['w'] + p['b']
    mu = h.mean(0, keepdims=True)
    var = ((h - mu) ** 2).mean(0, keepdims=True)
    return jax.nn.relu((h - mu) / jnp.sqrt(var + 1e-5))


def _genconv(p, x, src, dst, n):
    m = jax.nn.relu(x[src]) + 1e-7
    mt = m * p['t']
    mx = jax.ops.segment_max(mt, dst, num_segments=n)
    mx = jnp.where(jnp.isfinite(mx), mx, 0.0)
    e = jnp.exp(mt - mx[dst])
    s = jax.ops.segment_sum(e, dst, num_segments=n)
    alpha = e / (s[dst] + 1e-16)
    aggr = jax.ops.segment_sum(alpha * m, dst, num_segments=n)
    h = x + aggr
    h = jax.nn.relu(_layer_norm(h @ p['w1'] + p['b1']))
    return h @ p['w2'] + p['b2']


def _deepgcn(p, x, src, dst, n):
    h = jax.nn.relu(_layer_norm(x))
    return x + _genconv(p, h, src, dst, n)


def _mha(x, p, nh):
    b, s, d = x.shape
    dh = d // nh

    def sp(y):
        return y.reshape(b, s, nh, dh).transpose(0, 2, 1, 3)

    q = sp(x @ p['wq']['w'] + p['wq']['b'])
    k = sp(x @ p['wk']['w'] + p['wk']['b'])
    v = sp(x @ p['wv']['w'] + p['wv']['b'])
    a = jax.nn.softmax(q @ k.transpose(0, 1, 3, 2) / np.sqrt(dh), -1)
    o = (a @ v).transpose(0, 2, 1, 3).reshape(b, s, d)
    return o @ p['wo']['w'] + p['wo']['b']


def _tlayer(x, p, nh=2):
    x = _layer_norm(x + _mha(x, p, nh))
    f = jax.nn.relu(x @ p['f1']['w'] + p['f1']['b']) @ p['f2']['w'] + p['f2']['b']
    return _layer_norm(x + f)


def _spenc_feats(pos):
    scales = jnp.geomspace(1e-6, 8.0, 4).astype(jnp.float32)
    f = pos[:, None, :] / scales[None, :, None]
    return jnp.concatenate([jnp.sin(f), jnp.cos(f)], -1).reshape(pos.shape[0], -1)


def kernel(x, pos, batch, params):
    P = params
    levels, tds, ups = _build_structs(pos)
    emb = jax.nn.relu(_spenc_feats(pos) @ P['spenc']['w'] + P['spenc']['b'])
    emb = jnp.tanh(emb @ P['dec1']['w'] + P['dec1']['b'])
    emb = emb @ P['dec2']['w'] + P['dec2']['b']
    h = jnp.concatenate([x, emb], 1)
    h = _mlp_gn(h, P['mlp_in'])
    l0 = levels[0]
    h = _genconv(P['input_gc'][0], h, l0['src'], l0['dst'], l0['n'])
    h = _deepgcn(P['input_gc'][1], h, l0['src'], l0['dst'], l0['n'])
    outs = [h]
    for i in range(2):
        hh = _mlp_gn(h, P['td_mlp'][i])
        h = jax.ops.segment_max(hh[tds[i]['nbr']], tds[i]['seg'],
                                num_segments=tds[i]['m'])
        l = levels[i + 1]
        h = jax.nn.relu(_deepgcn(P['down_gc'][i], h, l['src'], l['dst'], l['n']))
        h = jax.nn.relu(_deepgcn(P['down_gc'][i], h, l['src'], l['dst'], l['n']))
        outs.append(h)
    xb = jax.nn.relu(h[None] @ P['mlp_summit']['w'] + P['mlp_summit']['b'])
    for tp in P['summit_tf']:
        xb = _tlayer(xb, tp)
    h = xb[0]
    for i in range(2):
        lvl = 1 - i
        up = ups[lvl]
        xs = _mlp_gn(h, P['tu_mlp_sub'][lvl])
        interp = jnp.sum(xs[up['idx']] * up['w'][..., None], 1)
        h = _mlp_gn(outs[lvl], P['tu_mlp'][lvl]) + interp
        l = levels[lvl]
        h = jax.nn.relu(_deepgcn(P['up_gc'][lvl], h, l['src'], l['dst'], l['n']))
        h = jax.nn.relu(_deepgcn(P['up_gc'][lvl], h, l['src'], l['dst'], l['n']))
    for gp in P['out_gc']:
        h = _deepgcn(gp, h, l0['src'], l0['dst'], l0['n'])
    return h @ P['mlp_out']['w'] + P['mlp_out']['b']


# speedup vs baseline: 3.5690x; 2.6413x over previous
import functools

import jax
import jax.numpy as jnp
import numpy as np
from jax.experimental import pallas as pl
from jax.experimental.pallas import tpu as pltpu

_DIM = [64, 128, 256]
_LIST_K = [10, 6, 3, 2]
_RATIO = 1.0 / 6.0
_K0 = 10


def _fps_body(px_ref, py_ref, pz_ref, o_ref, dist_ref, *, n, m, rows):
    iota = (jax.lax.broadcasted_iota(jnp.int32, (rows, 128), 0) * 128
            + jax.lax.broadcasted_iota(jnp.int32, (rows, 128), 1))
    valid = iota < n
    big = jnp.int32(2 ** 30)

    def point_at(fidx):
        mask = (iota == fidx)
        qx = jnp.sum(jnp.where(mask, px_ref[...], 0.0))
        qy = jnp.sum(jnp.where(mask, py_ref[...], 0.0))
        qz = jnp.sum(jnp.where(mask, pz_ref[...], 0.0))
        return qx, qy, qz

    def dist_to(qx, qy, qz):
        dx = px_ref[...] - qx
        dy = py_ref[...] - qy
        dz = pz_ref[...] - qz
        return dx * dx + dy * dy + dz * dz

    qx, qy, qz = point_at(jnp.int32(0))
    d0 = dist_to(qx, qy, qz)
    dist_ref[...] = jnp.where(valid, d0, -1.0)
    o_ref[0, 0] = jnp.int32(0)

    def body(i, _):
        dist = dist_ref[...]
        mval = jnp.max(dist)
        fidx = jnp.min(jnp.where(dist == mval, iota, big))
        o_ref[0, i] = fidx
        qx, qy, qz = point_at(fidx)
        nd = dist_to(qx, qy, qz)
        dist_ref[...] = jnp.minimum(dist, nd)
        return 0

    jax.lax.fori_loop(1, m, body, 0)


def _fps_pallas(pos, ratio):
    n = pos.shape[0]
    m = max(1, int(n * ratio))
    rows = -(-n // 128)
    npad = rows * 128
    posp = jnp.pad(pos, ((0, npad - n), (0, 0)))
    px = posp[:, 0].reshape(rows, 128)
    py = posp[:, 1].reshape(rows, 128)
    pz = posp[:, 2].reshape(rows, 128)
    sel = pl.pallas_call(
        functools.partial(_fps_body, n=n, m=m, rows=rows),
        in_specs=[pl.BlockSpec((rows, 128), lambda: (0, 0))] * 3,
        out_specs=pl.BlockSpec(memory_space=pltpu.SMEM),
        out_shape=jax.ShapeDtypeStruct((1, m), jnp.int32),
        scratch_shapes=[pltpu.VMEM((rows, 128), jnp.float32)],
    )(px, py, pz)
    return sel[0]


_BQ = 128


def _topk_body(qx_ref, qy_ref, qz_ref, bx_ref, by_ref, bz_ref, o_ref,
               dist_ref, *, k, bcols):
    dx = qx_ref[...] - bx_ref[...]
    dy = qy_ref[...] - by_ref[...]
    dz = qz_ref[...] - bz_ref[...]
    dist_ref[...] = dx * dx + dy * dy + dz * dz
    iota = jax.lax.broadcasted_iota(jnp.int32, (_BQ, bcols), 1)
    big = jnp.int32(2 ** 30)
    inf = jnp.float32(jnp.inf)
    for j in range(k):
        dist = dist_ref[...]
        m = jnp.min(dist, axis=1, keepdims=True)
        idx = jnp.min(jnp.where(dist == m, iota, big), axis=1, keepdims=True)
        o_ref[:, j:j + 1] = idx
        dist_ref[...] = jnp.where(iota == idx, inf, dist)


def _pairwise_topk(query, base, k):
    qn = query.shape[0]
    bn = base.shape[0]
    qpad = -(-qn // _BQ) * _BQ
    bpad = -(-bn // 128) * 128
    qp = jnp.pad(query, ((0, qpad - qn), (0, 0)))
    bp = jnp.pad(base, ((0, bpad - bn), (0, 0)), constant_values=1e6)
    qcols = [qp[:, i].reshape(qpad, 1) for i in range(3)]
    brows = [bp[:, i].reshape(1, bpad) for i in range(3)]
    grid = qpad // _BQ
    out = pl.pallas_call(
        functools.partial(_topk_body, k=k, bcols=bpad),
        grid=(grid,),
        in_specs=[pl.BlockSpec((_BQ, 1), lambda i: (i, 0))] * 3
        + [pl.BlockSpec((1, bpad), lambda i: (0, 0))] * 3,
        out_specs=pl.BlockSpec((_BQ, 128), lambda i: (i, 0)),
        out_shape=jax.ShapeDtypeStruct((qpad, 128), jnp.int32),
        scratch_shapes=[pltpu.VMEM((_BQ, bpad), jnp.float32)],
    )(*qcols, *brows)
    return out[:qn, :k]


def _knn_graph(pos, k):
    idx = _pairwise_topk(pos, pos, k + 1)[:, 1:]
    n = pos.shape[0]
    return idx.reshape(-1), jnp.repeat(jnp.arange(n), k)


def _build_structs(pos):
    src0, dst0 = _knn_graph(pos, _K0)
    levels = [{'pos': pos, 'src': src0, 'dst': dst0, 'n': pos.shape[0]}]
    tds = []
    cur = pos
    for i in range(2):
        idc = _fps_pallas(cur, _RATIO)
        m = idc.shape[0]
        sub = cur[idc]
        nbr = _pairwise_topk(sub, cur, _LIST_K[i])
        es, ed = _knn_graph(sub, _LIST_K[i + 1])
        tds.append({'nbr': nbr.reshape(-1),
                    'seg': jnp.repeat(jnp.arange(m), _LIST_K[i]), 'm': m})
        levels.append({'pos': sub, 'src': es, 'dst': ed, 'n': m})
        cur = sub
    ups = []
    for i in range(2):
        idx = _pairwise_topk(levels[i]['pos'], levels[i + 1]['pos'], 2)
        d = jnp.sum((levels[i]['pos'][:, None, :]
                     - levels[i + 1]['pos'][idx]) ** 2, -1)
        w = 1.0 / (d + 1e-8)
        w = w / jnp.sum(w, -1, keepdims=True)
        ups.append({'idx': idx, 'w': w})
    return levels, tds, ups


def _layer_norm(x):
    m = x.mean(-1, keepdims=True)
    v = ((x - m) ** 2).mean(-1, keepdims=True)
    return (x - m) / jnp.sqrt(v + 1e-5)


def _mlp_gn(x, p):
    h = x @ p['w'] + p['b']
    mu = h.mean(0, keepdims=True)
    var = ((h - mu) ** 2).mean(0, keepdims=True)
    return jax.nn.relu((h - mu) / jnp.sqrt(var + 1e-5))


def _genconv(p, x, src, dst, n):
    m = jax.nn.relu(x[src]) + 1e-7
    mt = m * p['t']
    mx = jax.ops.segment_max(mt, dst, num_segments=n)
    mx = jnp.where(jnp.isfinite(mx), mx, 0.0)
    e = jnp.exp(mt - mx[dst])
    s = jax.ops.segment_sum(e, dst, num_segments=n)
    alpha = e / (s[dst] + 1e-16)
    aggr = jax.ops.segment_sum(alpha * m, dst, num_segments=n)
    h = x + aggr
    h = jax.nn.relu(_layer_norm(h @ p['w1'] + p['b1']))
    return h @ p['w2'] + p['b2']


def _deepgcn(p, x, src, dst, n):
    h = jax.nn.relu(_layer_norm(x))
    return x + _genconv(p, h, src, dst, n)


def _mha(x, p, nh):
    b, s, d = x.shape
    dh = d // nh

    def sp(y):
        return y.reshape(b, s, nh, dh).transpose(0, 2, 1, 3)

    q = sp(x @ p['wq']['w'] + p['wq']['b'])
    k = sp(x @ p['wk']['w'] + p['wk']['b'])
    v = sp(x @ p['wv']['w'] + p['wv']['b'])
    a = jax.nn.softmax(q @ k.transpose(0, 1, 3, 2) / np.sqrt(dh), -1)
    o = (a @ v).transpose(0, 2, 1, 3).reshape(b, s, d)
    return o @ p['wo']['w'] + p['wo']['b']


def _tlayer(x, p, nh=2):
    x = _layer_norm(x + _mha(x, p, nh))
    f = jax.nn.relu(x @ p['f1']['w'] + p['f1']['b']) @ p['f2']['w'] + p['f2']['b']
    return _layer_norm(x + f)


def _spenc_feats(pos):
    scales = jnp.geomspace(1e-6, 8.0, 4).astype(jnp.float32)
    f = pos[:, None, :] / scales[None, :, None]
    return jnp.concatenate([jnp.sin(f), jnp.cos(f)], -1).reshape(pos.shape[0], -1)


def kernel(x, pos, batch, params):
    P = params
    levels, tds, ups = _build_structs(pos)
    emb = jax.nn.relu(_spenc_feats(pos) @ P['spenc']['w'] + P['spenc']['b'])
    emb = jnp.tanh(emb @ P['dec1']['w'] + P['dec1']['b'])
    emb = emb @ P['dec2']['w'] + P['dec2']['b']
    h = jnp.concatenate([x, emb], 1)
    h = _mlp_gn(h, P['mlp_in'])
    l0 = levels[0]
    h = _genconv(P['input_gc'][0], h, l0['src'], l0['dst'], l0['n'])
    h = _deepgcn(P['input_gc'][1], h, l0['src'], l0['dst'], l0['n'])
    outs = [h]
    for i in range(2):
        hh = _mlp_gn(h, P['td_mlp'][i])
        h = jax.ops.segment_max(hh[tds[i]['nbr']], tds[i]['seg'],
                                num_segments=tds[i]['m'])
        l = levels[i + 1]
        h = jax.nn.relu(_deepgcn(P['down_gc'][i], h, l['src'], l['dst'], l['n']))
        h = jax.nn.relu(_deepgcn(P['down_gc'][i], h, l['src'], l['dst'], l['n']))
        outs.append(h)
    xb = jax.nn.relu(h[None] @ P['mlp_summit']['w'] + P['mlp_summit']['b'])
    for tp in P['summit_tf']:
        xb = _tlayer(xb, tp)
    h = xb[0]
    for i in range(2):
        lvl = 1 - i
        up = ups[lvl]
        xs = _mlp_gn(h, P['tu_mlp_sub'][lvl])
        interp = jnp.sum(xs[up['idx']] * up['w'][..., None], 1)
        h = _mlp_gn(outs[lvl], P['tu_mlp'][lvl]) + interp
        l = levels[lvl]
        h = jax.nn.relu(_deepgcn(P['up_gc'][lvl], h, l['src'], l['dst'], l['n']))
        h = jax.nn.relu(_deepgcn(P['up_gc'][lvl], h, l['src'], l['dst'], l['n']))
    for gp in P['out_gc']:
        h = _deepgcn(gp, h, l0['src'], l0['dst'], l0['n'])
    return h @ P['mlp_out']['w'] + P['mlp_out']['b']


# speedup vs baseline: 9.2092x; 2.5803x over previous
import functools

import jax
import jax.numpy as jnp
import numpy as np
from jax.experimental import pallas as pl
from jax.experimental.pallas import tpu as pltpu

_DIM = [64, 128, 256]
_LIST_K = [10, 6, 3, 2]
_RATIO = 1.0 / 6.0
_K0 = 10


def _fps_body(px_ref, py_ref, pz_ref, o_ref, dist_ref, *, n, m, rows):
    iota = (jax.lax.broadcasted_iota(jnp.int32, (rows, 128), 0) * 128
            + jax.lax.broadcasted_iota(jnp.int32, (rows, 128), 1))
    valid = iota < n
    big = jnp.int32(2 ** 30)

    def point_at(fidx):
        mask = (iota == fidx)
        qx = jnp.sum(jnp.where(mask, px_ref[...], 0.0))
        qy = jnp.sum(jnp.where(mask, py_ref[...], 0.0))
        qz = jnp.sum(jnp.where(mask, pz_ref[...], 0.0))
        return qx, qy, qz

    def dist_to(qx, qy, qz):
        dx = px_ref[...] - qx
        dy = py_ref[...] - qy
        dz = pz_ref[...] - qz
        return dx * dx + dy * dy + dz * dz

    qx, qy, qz = point_at(jnp.int32(0))
    d0 = dist_to(qx, qy, qz)
    dist_ref[...] = jnp.where(valid, d0, -1.0)
    o_ref[0, 0] = jnp.int32(0)

    def body(i, _):
        dist = dist_ref[...]
        mval = jnp.max(dist)
        fidx = jnp.min(jnp.where(dist == mval, iota, big))
        o_ref[0, i] = fidx
        qx, qy, qz = point_at(fidx)
        nd = dist_to(qx, qy, qz)
        dist_ref[...] = jnp.minimum(dist, nd)
        return 0

    jax.lax.fori_loop(1, m, body, 0)


def _fps_pallas(pos, ratio):
    n = pos.shape[0]
    m = max(1, int(n * ratio))
    rows = -(-n // 128)
    npad = rows * 128
    posp = jnp.pad(pos, ((0, npad - n), (0, 0)))
    px = posp[:, 0].reshape(rows, 128)
    py = posp[:, 1].reshape(rows, 128)
    pz = posp[:, 2].reshape(rows, 128)
    sel = pl.pallas_call(
        functools.partial(_fps_body, n=n, m=m, rows=rows),
        in_specs=[pl.BlockSpec((rows, 128), lambda: (0, 0))] * 3,
        out_specs=pl.BlockSpec(memory_space=pltpu.SMEM),
        out_shape=jax.ShapeDtypeStruct((1, m), jnp.int32),
        scratch_shapes=[pltpu.VMEM((rows, 128), jnp.float32)],
    )(px, py, pz)
    return sel[0]


_BQ = 128


def _topk_body(qx_ref, qy_ref, qz_ref, bx_ref, by_ref, bz_ref, o_ref,
               dist_ref, *, k, bcols):
    dx = qx_ref[...] - bx_ref[...]
    dy = qy_ref[...] - by_ref[...]
    dz = qz_ref[...] - bz_ref[...]
    dist_ref[...] = dx * dx + dy * dy + dz * dz
    iota = jax.lax.broadcasted_iota(jnp.int32, (_BQ, bcols), 1)
    big = jnp.int32(2 ** 30)
    inf = jnp.float32(jnp.inf)
    for j in range(k):
        dist = dist_ref[...]
        m = jnp.min(dist, axis=1, keepdims=True)
        idx = jnp.min(jnp.where(dist == m, iota, big), axis=1, keepdims=True)
        o_ref[:, j:j + 1] = idx
        dist_ref[...] = jnp.where(iota == idx, inf, dist)


def _pairwise_topk(query, base, k):
    qn = query.shape[0]
    bn = base.shape[0]
    qpad = -(-qn // _BQ) * _BQ
    bpad = -(-bn // 128) * 128
    qp = jnp.pad(query, ((0, qpad - qn), (0, 0)))
    bp = jnp.pad(base, ((0, bpad - bn), (0, 0)), constant_values=1e6)
    qcols = [qp[:, i].reshape(qpad, 1) for i in range(3)]
    brows = [bp[:, i].reshape(1, bpad) for i in range(3)]
    grid = qpad // _BQ
    out = pl.pallas_call(
        functools.partial(_topk_body, k=k, bcols=bpad),
        grid=(grid,),
        in_specs=[pl.BlockSpec((_BQ, 1), lambda i: (i, 0))] * 3
        + [pl.BlockSpec((1, bpad), lambda i: (0, 0))] * 3,
        out_specs=pl.BlockSpec((_BQ, 128), lambda i: (i, 0)),
        out_shape=jax.ShapeDtypeStruct((qpad, 128), jnp.int32),
        scratch_shapes=[pltpu.VMEM((_BQ, bpad), jnp.float32)],
    )(*qcols, *brows)
    return out[:qn, :k]


def _knn_graph(pos, k):
    return _pairwise_topk(pos, pos, k + 1)[:, 1:]


def _build_structs(pos):
    levels = [{'pos': pos, 'idx': _knn_graph(pos, _K0), 'n': pos.shape[0]}]
    tds = []
    cur = pos
    for i in range(2):
        idc = _fps_pallas(cur, _RATIO)
        m = idc.shape[0]
        sub = cur[idc]
        nbr = _pairwise_topk(sub, cur, _LIST_K[i])
        es = _knn_graph(sub, _LIST_K[i + 1])
        tds.append({'nbr': nbr, 'm': m})
        levels.append({'pos': sub, 'idx': es, 'n': m})
        cur = sub
    ups = []
    for i in range(2):
        idx = _pairwise_topk(levels[i]['pos'], levels[i + 1]['pos'], 2)
        d = jnp.sum((levels[i]['pos'][:, None, :]
                     - levels[i + 1]['pos'][idx]) ** 2, -1)
        w = 1.0 / (d + 1e-8)
        w = w / jnp.sum(w, -1, keepdims=True)
        ups.append({'idx': idx, 'w': w})
    return levels, tds, ups


def _layer_norm(x):
    m = x.mean(-1, keepdims=True)
    v = ((x - m) ** 2).mean(-1, keepdims=True)
    return (x - m) / jnp.sqrt(v + 1e-5)


def _mlp_gn(x, p):
    h = x @ p['w'] + p['b']
    mu = h.mean(0, keepdims=True)
    var = ((h - mu) ** 2).mean(0, keepdims=True)
    return jax.nn.relu((h - mu) / jnp.sqrt(var + 1e-5))


def _kmax(a):
    r = a[:, 0]
    for j in range(1, a.shape[1]):
        r = jnp.maximum(r, a[:, j])
    return r


def _ksum(a):
    r = a[:, 0]
    for j in range(1, a.shape[1]):
        r = r + a[:, j]
    return r


def _genconv(p, x, idx):
    m = jax.nn.relu(x[idx]) + 1e-7
    mt = m * p['t']
    mx = _kmax(mt)
    e = jnp.exp(mt - mx[:, None, :])
    s = _ksum(e)
    alpha = e / (s[:, None, :] + 1e-16)
    aggr = _ksum(alpha * m)
    h = x + aggr
    h = jax.nn.relu(_layer_norm(h @ p['w1'] + p['b1']))
    return h @ p['w2'] + p['b2']


def _deepgcn(p, x, idx):
    h = jax.nn.relu(_layer_norm(x))
    return x + _genconv(p, h, idx)


def _mha(x, p, nh):
    b, s, d = x.shape
    dh = d // nh

    def sp(y):
        return y.reshape(b, s, nh, dh).transpose(0, 2, 1, 3)

    q = sp(x @ p['wq']['w'] + p['wq']['b'])
    k = sp(x @ p['wk']['w'] + p['wk']['b'])
    v = sp(x @ p['wv']['w'] + p['wv']['b'])
    a = jax.nn.softmax(q @ k.transpose(0, 1, 3, 2) / np.sqrt(dh), -1)
    o = (a @ v).transpose(0, 2, 1, 3).reshape(b, s, d)
    return o @ p['wo']['w'] + p['wo']['b']


def _tlayer(x, p, nh=2):
    x = _layer_norm(x + _mha(x, p, nh))
    f = jax.nn.relu(x @ p['f1']['w'] + p['f1']['b']) @ p['f2']['w'] + p['f2']['b']
    return _layer_norm(x + f)


def _spenc_feats(pos):
    scales = jnp.geomspace(1e-6, 8.0, 4).astype(jnp.float32)
    f = pos[:, None, :] / scales[None, :, None]
    return jnp.concatenate([jnp.sin(f), jnp.cos(f)], -1).reshape(pos.shape[0], -1)


def kernel(x, pos, batch, params):
    P = params
    levels, tds, ups = _build_structs(pos)
    emb = jax.nn.relu(_spenc_feats(pos) @ P['spenc']['w'] + P['spenc']['b'])
    emb = jnp.tanh(emb @ P['dec1']['w'] + P['dec1']['b'])
    emb = emb @ P['dec2']['w'] + P['dec2']['b']
    h = jnp.concatenate([x, emb], 1)
    h = _mlp_gn(h, P['mlp_in'])
    l0 = levels[0]
    h = _genconv(P['input_gc'][0], h, l0['idx'])
    h = _deepgcn(P['input_gc'][1], h, l0['idx'])
    outs = [h]
    for i in range(2):
        hh = _mlp_gn(h, P['td_mlp'][i])
        h = _kmax(hh[tds[i]['nbr']])
        l = levels[i + 1]
        h = jax.nn.relu(_deepgcn(P['down_gc'][i], h, l['idx']))
        h = jax.nn.relu(_deepgcn(P['down_gc'][i], h, l['idx']))
        outs.append(h)
    xb = jax.nn.relu(h[None] @ P['mlp_summit']['w'] + P['mlp_summit']['b'])
    for tp in P['summit_tf']:
        xb = _tlayer(xb, tp)
    h = xb[0]
    for i in range(2):
        lvl = 1 - i
        up = ups[lvl]
        xs = _mlp_gn(h, P['tu_mlp_sub'][lvl])
        interp = jnp.sum(xs[up['idx']] * up['w'][..., None], 1)
        h = _mlp_gn(outs[lvl], P['tu_mlp'][lvl]) + interp
        l = levels[lvl]
        h = jax.nn.relu(_deepgcn(P['up_gc'][lvl], h, l['idx']))
        h = jax.nn.relu(_deepgcn(P['up_gc'][lvl], h, l['idx']))
    for gp in P['out_gc']:
        h = _deepgcn(gp, h, l0['idx'])
    return h @ P['mlp_out']['w'] + P['mlp_out']['b']


# speedup vs baseline: 9.2124x; 1.0003x over previous
import functools

import jax
import jax.numpy as jnp
import numpy as np
from jax.experimental import pallas as pl
from jax.experimental.pallas import tpu as pltpu

_DIM = [64, 128, 256]
_LIST_K = [10, 6, 3, 2]
_RATIO = 1.0 / 6.0
_K0 = 10


def _fps_body(px_ref, py_ref, pz_ref, o_ref, dist_ref, *, n, m, rows):
    iota = (jax.lax.broadcasted_iota(jnp.int32, (rows, 128), 0) * 128
            + jax.lax.broadcasted_iota(jnp.int32, (rows, 128), 1))
    valid = iota < n
    big = jnp.int32(2 ** 30)

    lane_iota = jax.lax.broadcasted_iota(jnp.int32, (1, 128), 1)

    def point_at(fidx):
        row = fidx // 128
        col = fidx % 128
        lmask = lane_iota == col
        qx = jnp.sum(jnp.where(lmask, px_ref[pl.ds(row, 1), :], 0.0))
        qy = jnp.sum(jnp.where(lmask, py_ref[pl.ds(row, 1), :], 0.0))
        qz = jnp.sum(jnp.where(lmask, pz_ref[pl.ds(row, 1), :], 0.0))
        return qx, qy, qz

    def dist_to(qx, qy, qz):
        dx = px_ref[...] - qx
        dy = py_ref[...] - qy
        dz = pz_ref[...] - qz
        return dx * dx + dy * dy + dz * dz

    qx, qy, qz = point_at(jnp.int32(0))
    d0 = dist_to(qx, qy, qz)
    dist_ref[...] = jnp.where(valid, d0, -1.0)
    o_ref[0, 0] = jnp.int32(0)

    def body(i, _):
        dist = dist_ref[...]
        mval = jnp.max(dist)
        fidx = jnp.min(jnp.where(dist == mval, iota, big))
        o_ref[0, i] = fidx
        qx, qy, qz = point_at(fidx)
        nd = dist_to(qx, qy, qz)
        dist_ref[...] = jnp.minimum(dist, nd)
        return 0

    jax.lax.fori_loop(1, m, body, 0)


def _fps_pallas(pos, ratio):
    n = pos.shape[0]
    m = max(1, int(n * ratio))
    rows = -(-n // 128)
    npad = rows * 128
    posp = jnp.pad(pos, ((0, npad - n), (0, 0)))
    px = posp[:, 0].reshape(rows, 128)
    py = posp[:, 1].reshape(rows, 128)
    pz = posp[:, 2].reshape(rows, 128)
    sel = pl.pallas_call(
        functools.partial(_fps_body, n=n, m=m, rows=rows),
        in_specs=[pl.BlockSpec((rows, 128), lambda: (0, 0))] * 3,
        out_specs=pl.BlockSpec(memory_space=pltpu.SMEM),
        out_shape=jax.ShapeDtypeStruct((1, m), jnp.int32),
        scratch_shapes=[pltpu.VMEM((rows, 128), jnp.float32)],
    )(px, py, pz)
    return sel[0]


_BQ = 128


def _topk_body(qx_ref, qy_ref, qz_ref, bx_ref, by_ref, bz_ref, o_ref,
               dist_ref, *, k, bcols):
    dx = qx_ref[...] - bx_ref[...]
    dy = qy_ref[...] - by_ref[...]
    dz = qz_ref[...] - bz_ref[...]
    dist_ref[...] = dx * dx + dy * dy + dz * dz
    iota = jax.lax.broadcasted_iota(jnp.int32, (_BQ, bcols), 1)
    big = jnp.int32(2 ** 30)
    inf = jnp.float32(jnp.inf)
    for j in range(k):
        dist = dist_ref[...]
        m = jnp.min(dist, axis=1, keepdims=True)
        idx = jnp.min(jnp.where(dist == m, iota, big), axis=1, keepdims=True)
        o_ref[:, j:j + 1] = idx
        dist_ref[...] = jnp.where(iota == idx, inf, dist)


def _pairwise_topk(query, base, k):
    qn = query.shape[0]
    bn = base.shape[0]
    qpad = -(-qn // _BQ) * _BQ
    bpad = -(-bn // 128) * 128
    qp = jnp.pad(query, ((0, qpad - qn), (0, 0)))
    bp = jnp.pad(base, ((0, bpad - bn), (0, 0)), constant_values=1e6)
    qcols = [qp[:, i].reshape(qpad, 1) for i in range(3)]
    brows = [bp[:, i].reshape(1, bpad) for i in range(3)]
    grid = qpad // _BQ
    out = pl.pallas_call(
        functools.partial(_topk_body, k=k, bcols=bpad),
        grid=(grid,),
        in_specs=[pl.BlockSpec((_BQ, 1), lambda i: (i, 0))] * 3
        + [pl.BlockSpec((1, bpad), lambda i: (0, 0))] * 3,
        out_specs=pl.BlockSpec((_BQ, 128), lambda i: (i, 0)),
        out_shape=jax.ShapeDtypeStruct((qpad, 128), jnp.int32),
        scratch_shapes=[pltpu.VMEM((_BQ, bpad), jnp.float32)],
    )(*qcols, *brows)
    return out[:qn, :k]


def _knn_graph(pos, k):
    return _pairwise_topk(pos, pos, k + 1)[:, 1:]


def _build_structs(pos):
    levels = [{'pos': pos, 'idx': _knn_graph(pos, _K0), 'n': pos.shape[0]}]
    tds = []
    cur = pos
    for i in range(2):
        idc = _fps_pallas(cur, _RATIO)
        m = idc.shape[0]
        sub = cur[idc]
        nbr = _pairwise_topk(sub, cur, _LIST_K[i])
        es = _knn_graph(sub, _LIST_K[i + 1])
        tds.append({'nbr': nbr, 'm': m})
        levels.append({'pos': sub, 'idx': es, 'n': m})
        cur = sub
    ups = []
    for i in range(2):
        idx = _pairwise_topk(levels[i]['pos'], levels[i + 1]['pos'], 2)
        d = jnp.sum((levels[i]['pos'][:, None, :]
                     - levels[i + 1]['pos'][idx]) ** 2, -1)
        w = 1.0 / (d + 1e-8)
        w = w / jnp.sum(w, -1, keepdims=True)
        ups.append({'idx': idx, 'w': w})
    return levels, tds, ups


def _layer_norm(x):
    m = x.mean(-1, keepdims=True)
    v = ((x - m) ** 2).mean(-1, keepdims=True)
    return (x - m) / jnp.sqrt(v + 1e-5)


def _mlp_gn(x, p):
    h = x @ p['w'] + p['b']
    mu = h.mean(0, keepdims=True)
    var = ((h - mu) ** 2).mean(0, keepdims=True)
    return jax.nn.relu((h - mu) / jnp.sqrt(var + 1e-5))


def _kmax(a):
    r = a[:, 0]
    for j in range(1, a.shape[1]):
        r = jnp.maximum(r, a[:, j])
    return r


def _ksum(a):
    r = a[:, 0]
    for j in range(1, a.shape[1]):
        r = r + a[:, j]
    return r


def _genconv(p, x, idx):
    m = jax.nn.relu(x[idx]) + 1e-7
    mt = m * p['t']
    mx = _kmax(mt)
    e = jnp.exp(mt - mx[:, None, :])
    s = _ksum(e)
    alpha = e / (s[:, None, :] + 1e-16)
    aggr = _ksum(alpha * m)
    h = x + aggr
    h = jax.nn.relu(_layer_norm(h @ p['w1'] + p['b1']))
    return h @ p['w2'] + p['b2']


def _deepgcn(p, x, idx):
    h = jax.nn.relu(_layer_norm(x))
    return x + _genconv(p, h, idx)


def _mha(x, p, nh):
    b, s, d = x.shape
    dh = d // nh

    def sp(y):
        return y.reshape(b, s, nh, dh).transpose(0, 2, 1, 3)

    q = sp(x @ p['wq']['w'] + p['wq']['b'])
    k = sp(x @ p['wk']['w'] + p['wk']['b'])
    v = sp(x @ p['wv']['w'] + p['wv']['b'])
    a = jax.nn.softmax(q @ k.transpose(0, 1, 3, 2) / np.sqrt(dh), -1)
    o = (a @ v).transpose(0, 2, 1, 3).reshape(b, s, d)
    return o @ p['wo']['w'] + p['wo']['b']


def _tlayer(x, p, nh=2):
    x = _layer_norm(x + _mha(x, p, nh))
    f = jax.nn.relu(x @ p['f1']['w'] + p['f1']['b']) @ p['f2']['w'] + p['f2']['b']
    return _layer_norm(x + f)


def _spenc_feats(pos):
    scales = jnp.geomspace(1e-6, 8.0, 4).astype(jnp.float32)
    f = pos[:, None, :] / scales[None, :, None]
    return jnp.concatenate([jnp.sin(f), jnp.cos(f)], -1).reshape(pos.shape[0], -1)


def kernel(x, pos, batch, params):
    P = params
    levels, tds, ups = _build_structs(pos)
    emb = jax.nn.relu(_spenc_feats(pos) @ P['spenc']['w'] + P['spenc']['b'])
    emb = jnp.tanh(emb @ P['dec1']['w'] + P['dec1']['b'])
    emb = emb @ P['dec2']['w'] + P['dec2']['b']
    h = jnp.concatenate([x, emb], 1)
    h = _mlp_gn(h, P['mlp_in'])
    l0 = levels[0]
    h = _genconv(P['input_gc'][0], h, l0['idx'])
    h = _deepgcn(P['input_gc'][1], h, l0['idx'])
    outs = [h]
    for i in range(2):
        hh = _mlp_gn(h, P['td_mlp'][i])
        h = _kmax(hh[tds[i]['nbr']])
        l = levels[i + 1]
        h = jax.nn.relu(_deepgcn(P['down_gc'][i], h, l['idx']))
        h = jax.nn.relu(_deepgcn(P['down_gc'][i], h, l['idx']))
        outs.append(h)
    xb = jax.nn.relu(h[None] @ P['mlp_summit']['w'] + P['mlp_summit']['b'])
    for tp in P['summit_tf']:
        xb = _tlayer(xb, tp)
    h = xb[0]
    for i in range(2):
        lvl = 1 - i
        up = ups[lvl]
        xs = _mlp_gn(h, P['tu_mlp_sub'][lvl])
        interp = jnp.sum(xs[up['idx']] * up['w'][..., None], 1)
        h = _mlp_gn(outs[lvl], P['tu_mlp'][lvl]) + interp
        l = levels[lvl]
        h = jax.nn.relu(_deepgcn(P['up_gc'][lvl], h, l['idx']))
        h = jax.nn.relu(_deepgcn(P['up_gc'][lvl], h, l['idx']))
    for gp in P['out_gc']:
        h = _deepgcn(gp, h, l0['idx'])
    return h @ P['mlp_out']['w'] + P['mlp_out']['b']
